# baseline (speedup 1.0000x reference)
import functools

import jax
import jax.numpy as jnp
from jax import lax
from jax.experimental import pallas as pl
from jax.experimental.pallas import tpu as pltpu
from jax.experimental.pallas import tpu_sc as plsc

_NC = 2
_NS = 16
_NW = _NC * _NS
_IW = 128


@functools.partial(jax.jit, static_argnames=("n_rows", "dim", "steps", "chunks"))
def _gather_call(idx, table, n_rows, dim, steps, chunks):
    chunk_rows = steps * _IW
    mesh = plsc.VectorSubcoreMesh(core_axis_name="c", subcore_axis_name="s")

    @functools.partial(
        pl.kernel,
        mesh=mesh,
        compiler_params=pltpu.CompilerParams(use_tc_tiling_on_sc=False),
        out_type=jax.ShapeDtypeStruct((n_rows, dim), jnp.float32),
        scratch_types=[
            pltpu.VMEM((steps, _IW), jnp.int32),
            pltpu.VMEM((chunk_rows, dim), jnp.float32),
            pltpu.SemaphoreType.DMA,
        ],
    )
    def emb(idx_hbm, tbl_hbm, out_hbm, idx_v, rows_v, sem):
        wid = lax.axis_index("s") * _NC + lax.axis_index("c")
        idx_row0 = wid * (chunks * steps)
        out_row0 = wid * (chunks * chunk_rows)

        def body(g, carry):
            pltpu.sync_copy(idx_hbm.at[pl.ds(idx_row0 + g * steps, steps)], idx_v)
            copies = [
                pltpu.async_copy(
                    tbl_hbm.at[idx_v.at[j]],
                    rows_v.at[pl.ds(j * _IW, _IW)],
                    sem,
                )
                for j in range(steps)
            ]
            for c in copies:
                c.wait()
            pltpu.sync_copy(
                rows_v, out_hbm.at[pl.ds(out_row0 + g * chunk_rows, chunk_rows)]
            )
            return carry

        lax.fori_loop(0, chunks, body, 0)

    return emb(idx, table)


def kernel(input_ids, table):
    batch, hist = input_ids.shape
    vocab, dim = table.shape
    n = batch * hist

    steps = 8
    chunk_rows = steps * _IW
    assert n % (_NW * chunk_rows) == 0
    chunks = n // (_NW * chunk_rows)

    idx = input_ids.reshape(n // _IW, _IW).astype(jnp.int32)
    out = _gather_call(idx, table, n, dim, steps, chunks)
    return out.reshape(batch, hist, dim)


# speedup vs baseline: 1.0121x; 1.0121x over previous
import functools

import jax
import jax.numpy as jnp
from jax import lax
from jax.experimental import pallas as pl
from jax.experimental.pallas import tpu as pltpu
from jax.experimental.pallas import tpu_sc as plsc

_NC = 2
_NS = 16
_NW = _NC * _NS
_IW = 128


@functools.partial(jax.jit, static_argnames=("n_rows", "dim", "steps", "chunks"))
def _gather_call(idx, table, n_rows, dim, steps, chunks):
    cr = steps * _IW
    mesh = plsc.VectorSubcoreMesh(core_axis_name="c", subcore_axis_name="s")

    @functools.partial(
        pl.kernel,
        mesh=mesh,
        compiler_params=pltpu.CompilerParams(use_tc_tiling_on_sc=False),
        out_type=jax.ShapeDtypeStruct((n_rows, dim), jnp.float32),
        scratch_types=[
            pltpu.VMEM((steps, _IW), jnp.int32),
            pltpu.VMEM((steps, _IW), jnp.int32),
            pltpu.VMEM((cr, dim), jnp.float32),
            pltpu.VMEM((cr, dim), jnp.float32),
            pltpu.SemaphoreType.DMA,
            pltpu.SemaphoreType.DMA,
            pltpu.SemaphoreType.DMA,
            pltpu.SemaphoreType.DMA,
        ],
    )
    def emb(idx_hbm, tbl_hbm, out_hbm,
            idx0, idx1, rows0, rows1, gsem0, gsem1, wsem0, wsem1):
        wid = lax.axis_index("s") * _NC + lax.axis_index("c")
        idx_row0 = wid * (chunks * steps)
        out_row0 = wid * (chunks * cr)
        idxs, rows = (idx0, idx1), (rows0, rows1)
        gsems, wsems = (gsem0, gsem1), (wsem0, wsem1)

        def fire_gathers(g, b):
            pltpu.sync_copy(idx_hbm.at[pl.ds(idx_row0 + g * steps, steps)],
                            idxs[b])
            for j in range(steps):
                pltpu.async_copy(tbl_hbm.at[idxs[b].at[j]],
                                 rows[b].at[pl.ds(j * _IW, _IW)], gsems[b])

        def drain_gathers(b):
            pltpu.make_async_copy(out_hbm.at[pl.ds(0, cr)], rows[b],
                                  gsems[b]).wait()

        def fire_write(g, b):
            pltpu.async_copy(rows[b], out_hbm.at[pl.ds(out_row0 + g * cr, cr)],
                             wsems[b])

        def drain_write(b):
            pltpu.make_async_copy(rows[b], out_hbm.at[pl.ds(0, cr)],
                                  wsems[b]).wait()

        def section(g, b, h_ge_1):
            def finish_prev():
                drain_gathers(1 - b)
                fire_write(g - 1, 1 - b)
            if b == 0:
                pl.when(h_ge_1)(finish_prev)
                pl.when(h_ge_1)(lambda: drain_write(b))
            else:
                finish_prev()
                pl.when(h_ge_1)(lambda: drain_write(b))
            fire_gathers(g, b)

        def body(h, carry):
            h_ge_1 = h >= 1
            section(2 * h, 0, h_ge_1)
            section(2 * h + 1, 1, h_ge_1)
            return carry

        lax.fori_loop(0, chunks // 2, body, 0)
        drain_gathers(1)
        fire_write(chunks - 1, 1)
        drain_write(0)
        drain_write(1)

    return emb(idx, table)


def kernel(input_ids, table):
    batch, hist = input_ids.shape
    vocab, dim = table.shape
    n = batch * hist

    steps = 4
    cr = steps * _IW
    assert n % (_NW * cr * 2) == 0
    chunks = n // (_NW * cr)

    idx = input_ids.reshape(n // _IW, _IW).astype(jnp.int32)
    out = _gather_call(idx, table, n, dim, steps, chunks)
    return out.reshape(batch, hist, dim)
